# baseline (speedup 1.0000x reference)
import functools

import numpy as np
import jax
import jax.numpy as jnp
from jax import lax
from jax.experimental import pallas as pl
from jax.experimental.pallas import tpu as pltpu
from jax.experimental.pallas import tpu_sc as plsc

_VOCAB = 1000000
_D = 64
_SEQ = 200
_BATCH = 4096

_NC = 2
_NS = 16
_NW = _NC * _NS

_IDX_MINOR = 100
_SUBS = 4
_CHUNK = _IDX_MINOR * _SUBS
_TOTAL = _BATCH * _SEQ
_PER_W = _TOTAL // _NW
_CHUNKS = _PER_W // _CHUNK
_IDX_ROWS = _TOTAL // _IDX_MINOR


def _pos_encoding(length, depth):
    d = depth // 2
    positions = np.arange(length)[:, np.newaxis]
    depths = np.arange(d)[np.newaxis, :] / d
    rads = positions / 10000 ** depths
    pe = np.concatenate([np.sin(rads), np.cos(rads)], axis=-1)
    return jnp.asarray(pe, dtype=jnp.float32)


def _sc_embed(table, idx, pe):
    mesh = plsc.VectorSubcoreMesh(core_axis_name="c", subcore_axis_name="s")

    @functools.partial(
        pl.kernel,
        mesh=mesh,
        compiler_params=pltpu.CompilerParams(use_tc_tiling_on_sc=False),
        out_type=jax.ShapeDtypeStruct((_TOTAL, _D), jnp.float32),
        scratch_types=[
            pltpu.VMEM((_SUBS, _IDX_MINOR), jnp.int32),
            pltpu.VMEM((_CHUNK, _D), jnp.float32),
            pltpu.VMEM((_SEQ, _D), jnp.float32),
            pltpu.SemaphoreType.DMA,
        ],
    )
    def k(table_hbm, idx_hbm, pe_hbm, out_hbm, idx_v, gbuf, pe_v, sem):
        wid = lax.axis_index("s") * _NC + lax.axis_index("c")
        pltpu.sync_copy(pe_hbm, pe_v)

        def chunk_body(g, carry):
            row0 = wid * (_PER_W // _IDX_MINOR) + g * _SUBS
            pltpu.sync_copy(idx_hbm.at[pl.ds(row0, _SUBS)], idx_v)
            cps = [
                pltpu.async_copy(
                    table_hbm.at[idx_v.at[j]],
                    gbuf.at[pl.ds(j * _IDX_MINOR, _IDX_MINOR)],
                    sem,
                )
                for j in range(_SUBS)
            ]
            for cp in cps:
                cp.wait()

            def pe_body(r, c):
                for gg in range(_D // 16):
                    v = pe_v[r, pl.ds(gg * 16, 16)]
                    plsc.addupdate(gbuf.at[r, pl.ds(gg * 16, 16)], v)
                    plsc.addupdate(gbuf.at[r + _SEQ, pl.ds(gg * 16, 16)], v)
                return c

            lax.fori_loop(0, _SEQ, pe_body, 0)
            pltpu.sync_copy(
                gbuf, out_hbm.at[pl.ds(wid * _PER_W + g * _CHUNK, _CHUNK)]
            )
            return carry

        lax.fori_loop(0, _CHUNKS, chunk_body, 0)

    return k(table, idx, pe)


def kernel(x, table):
    b, s = x.shape
    idx = x.astype(jnp.int32).reshape(_IDX_ROWS, _IDX_MINOR)
    pe = _pos_encoding(s, table.shape[1])
    out = _sc_embed(table, idx, pe)
    return out.reshape(b, s, table.shape[1])
